# Initial kernel scaffold; baseline (speedup 1.0000x reference)
import jax
import jax.numpy as jnp
from jax.experimental import pallas as pl


def kernel(x, edge_index, edge_attr, batch, y, ee0_W, ee0_b, pre0_W, pre0_b, post0_W, post0_b, lin0_W, lin0_b, ee1_W, ee1_b, pre1_W, pre1_b, post1_W, post1_b, lin1_W, lin1_b, fm_W, fm_b, m1_W, m1_b, m2_W, m2_b, m3_W, m3_b, m4_W, m4_b):
    raise NotImplementedError("write your pallas kernel here")


# speedup vs baseline: 15.4777x; 15.4777x over previous
import functools
import numpy as np
import jax
import jax.numpy as jnp
from jax import lax
from jax.experimental import pallas as pl
from jax.experimental.pallas import tpu as pltpu
from jax.experimental.pallas import tpu_sc as plsc

N_NODES = 10000
N_EDGES = 160000
TOWERS = 5
NUM_GRAPHS = 64
AVG_LOG = float(np.log(33.0))

SC_NC = 2
SC_NS = 16
SC_NW = SC_NC * SC_NS

NB = 25
NBLK = N_NODES // NB
KE = 16
BIG = 3.0e38


BPW = 13
NSP = 432


def _sc_edge_kernel(TG):
    NV = TG // 16
    mesh = plsc.VectorSubcoreMesh(core_axis_name="c", subcore_axis_name="s")

    @functools.partial(
        pl.kernel,
        mesh=mesh,
        out_type=[
            jax.ShapeDtypeStruct((N_NODES * TG,), jnp.float32),
            jax.ShapeDtypeStruct((N_NODES * TG,), jnp.float32),
            jax.ShapeDtypeStruct((N_NODES * TG,), jnp.float32),
            jax.ShapeDtypeStruct((N_NODES * TG,), jnp.float32),
            jax.ShapeDtypeStruct((N_NODES * 16,), jnp.float32),
        ],
        scratch_types=[
            pltpu.VMEM(((NB + 1) * TG,), jnp.float32),
            pltpu.VMEM(((NB + 1) * TG,), jnp.float32),
            pltpu.VMEM(((NB + 1) * TG,), jnp.float32),
            pltpu.VMEM(((NB + 1) * TG,), jnp.float32),
            pltpu.VMEM(((NB + 1) * 16,), jnp.float32),
            pltpu.VMEM((4, TG), jnp.float32),
            pltpu.VMEM((TG,), jnp.float32),
            pltpu.VMEM((NSP,), jnp.int32),
            pltpu.VMEM((KE,), jnp.int32),
            pltpu.VMEM((KE,), jnp.int32),
            pltpu.VMEM((KE, 16), jnp.float32),
            pltpu.VMEM((KE, TG), jnp.float32),
            pltpu.SemaphoreType.DMA,
        ],
    )
    def kern(u_hbm, ssrc_hbm, sdst_hbm, sattr16_hbm, cw_hbm, cb_hbm,
             starts_hbm, s1_hbm, s2_hbm, mn_hbm, mx_hbm, deg_hbm,
             acc1, acc2, accmn, accmx, accdeg, cw_v, cb_v, starts_v,
             src_v, dst_v, attr_v, gat_v, sem):
        wid = lax.axis_index("s") * SC_NC + lax.axis_index("c")
        pltpu.sync_copy(cw_hbm, cw_v)
        pltpu.sync_copy(cb_hbm, cb_v)
        pltpu.sync_copy(starts_hbm, starts_v)
        iota16 = lax.iota(jnp.int32, 16)
        basis = jnp.where(iota16 == 0, 1.0, 0.0).astype(jnp.float32)
        zeros16 = jnp.zeros((16,), jnp.float32)
        big16 = jnp.full((16,), BIG, jnp.float32)

        def do_block(blk):
            bnode = blk * NB
            ev = starts_v[pl.ds(blk, 16)]
            e0 = ev[0]
            e1 = ev[1]

            def initj(r, _):
                sl = pl.ds(r * 16, 16)
                acc1[sl] = zeros16
                acc2[sl] = zeros16
                accmn[sl] = big16
                accmx[sl] = -big16
                return 0
            lax.fori_loop(0, (NB + 1) * NV, initj, 0)

            def initd(r, _):
                accdeg[pl.ds(r * 16, 16)] = zeros16
                return 0
            lax.fori_loop(0, NB + 1, initd, 0)

            c0 = lax.div(e0, KE)
            c1 = lax.div(e1 + (KE - 1), KE)

            def chunkbody(ci, _):
                c = ci * KE
                pltpu.sync_copy(ssrc_hbm.at[pl.ds(c, KE)], src_v)
                pltpu.sync_copy(sdst_hbm.at[pl.ds(c, KE)], dst_v)
                pltpu.sync_copy(sattr16_hbm.at[pl.ds(c, KE)], attr_v)
                pltpu.async_copy(u_hbm.at[src_v], gat_v, sem).wait()
                dstv = dst_v[pl.ds(0, KE)]
                for i in range(KE):
                    local = dstv[i] - bnode
                    valid = jnp.logical_and(local >= 0, local < NB)
                    lidx = jnp.where(valid, local, NB)
                    base = lidx * TG
                    av = attr_v[i, pl.ds(0, 16)]
                    a0 = av[0]
                    a1 = av[1]
                    a2 = av[2]
                    a3 = av[3]

                    def jbody(j, _):
                        sl = pl.ds(j * 16, 16)
                        asl = pl.ds(base + j * 16, 16)
                        b = (gat_v[i, sl] + a0 * cw_v[0, sl]
                             + a1 * cw_v[1, sl] + a2 * cw_v[2, sl]
                             + a3 * cw_v[3, sl] + cb_v[sl])
                        plsc.addupdate(acc1.at[asl], b)
                        plsc.addupdate(acc2.at[asl], b * b)
                        accmn[asl] = jnp.minimum(accmn[asl], b)
                        accmx[asl] = jnp.maximum(accmx[asl], b)
                        return 0
                    lax.fori_loop(0, NV, jbody, 0)
                    plsc.addupdate(accdeg.at[pl.ds(lidx * 16, 16)], basis)
                return 0
            lax.fori_loop(c0, c1, chunkbody, 0)

            ob = bnode * TG
            pltpu.sync_copy(acc1.at[pl.ds(0, NB * TG)],
                            s1_hbm.at[pl.ds(ob, NB * TG)])
            pltpu.sync_copy(acc2.at[pl.ds(0, NB * TG)],
                            s2_hbm.at[pl.ds(ob, NB * TG)])
            pltpu.sync_copy(accmn.at[pl.ds(0, NB * TG)],
                            mn_hbm.at[pl.ds(ob, NB * TG)])
            pltpu.sync_copy(accmx.at[pl.ds(0, NB * TG)],
                            mx_hbm.at[pl.ds(ob, NB * TG)])
            pltpu.sync_copy(accdeg.at[pl.ds(0, NB * 16)],
                            deg_hbm.at[pl.ds(bnode * 16, NB * 16)])

        def wbody(k, _):
            blk = wid * BPW + k

            @pl.when(blk < NBLK)
            def _():
                do_block(blk)
            return 0
        lax.fori_loop(0, BPW, wbody, 0)

    return kern


def _mm(x, w, bn):
    n, f = x.shape
    m = w.shape[1]

    def body(x_ref, w_ref, o_ref):
        o_ref[...] = jnp.dot(x_ref[...], w_ref[...],
                             preferred_element_type=jnp.float32)

    return pl.pallas_call(
        body,
        grid=(n // bn,),
        in_specs=[pl.BlockSpec((bn, f), lambda i: (i, 0)),
                  pl.BlockSpec((f, m), lambda i: (0, 0))],
        out_specs=pl.BlockSpec((bn, m), lambda i: (i, 0)),
        out_shape=jax.ShapeDtypeStruct((n, m), jnp.float32),
    )(x, w)


def _node_phase(x, s1, s2, mn, mx, deg, wd2, pxf, b1s, b2s, b3s, postb_f,
                lin_w, lin_b, relu, bn):
    n, fin = x.shape
    tg = s1.shape[1]
    emb = lin_w.shape[0]

    def body(x_ref, s1_ref, s2_ref, mn_ref, mx_ref, deg_ref,
             wd2_ref, pxf_ref, b1_ref, b2_ref, b3_ref, pb_ref,
             lw_ref, lb_ref, o_ref):
        xb = x_ref[...]
        a = jnp.dot(xb, wd2_ref[...], preferred_element_type=jnp.float32)
        deg_b = deg_ref[:, 0:1]
        degc = jnp.maximum(deg_b, 1.0)
        s1v = s1_ref[...]
        s2v = s2_ref[...]
        inv = 1.0 / degc
        mean = (deg_b * a + s1v) * inv
        mean2 = (deg_b * a * a + 2.0 * a * s1v + s2v) * inv
        std = jnp.sqrt(jax.nn.relu(mean2 - mean * mean) + 1e-5)
        pos = deg_b > 0.0
        mnv = jnp.where(pos, a + mn_ref[...], 0.0)
        mxv = jnp.where(pos, a + mx_ref[...], 0.0)
        degl = jnp.log(degc + 1.0)
        sc1 = degl * (1.0 / AVG_LOG)
        sc2 = AVG_LOG / degl
        r1 = (jnp.dot(mean, b1_ref[0], preferred_element_type=jnp.float32)
              + jnp.dot(mnv, b1_ref[1], preferred_element_type=jnp.float32)
              + jnp.dot(mxv, b1_ref[2], preferred_element_type=jnp.float32)
              + jnp.dot(std, b1_ref[3], preferred_element_type=jnp.float32))
        r2 = (jnp.dot(mean, b2_ref[0], preferred_element_type=jnp.float32)
              + jnp.dot(mnv, b2_ref[1], preferred_element_type=jnp.float32)
              + jnp.dot(mxv, b2_ref[2], preferred_element_type=jnp.float32)
              + jnp.dot(std, b2_ref[3], preferred_element_type=jnp.float32))
        r3 = (jnp.dot(mean, b3_ref[0], preferred_element_type=jnp.float32)
              + jnp.dot(mnv, b3_ref[1], preferred_element_type=jnp.float32)
              + jnp.dot(mxv, b3_ref[2], preferred_element_type=jnp.float32)
              + jnp.dot(std, b3_ref[3], preferred_element_type=jnp.float32))
        r = (jnp.dot(xb, pxf_ref[...], preferred_element_type=jnp.float32)
             + r1 + sc1 * r2 + sc2 * r3 + pb_ref[...])
        h = jnp.dot(r, lw_ref[...], preferred_element_type=jnp.float32) \
            + lb_ref[...]
        if relu:
            h = jax.nn.relu(h)
        o_ref[...] = h

    wfull = lambda shape: pl.BlockSpec(shape, lambda i: tuple(0 for _ in shape))
    return pl.pallas_call(
        body,
        grid=(n // bn,),
        in_specs=[
            pl.BlockSpec((bn, fin), lambda i: (i, 0)),
            pl.BlockSpec((bn, tg), lambda i: (i, 0)),
            pl.BlockSpec((bn, tg), lambda i: (i, 0)),
            pl.BlockSpec((bn, tg), lambda i: (i, 0)),
            pl.BlockSpec((bn, tg), lambda i: (i, 0)),
            pl.BlockSpec((bn, 16), lambda i: (i, 0)),
            wfull((fin, tg)),
            wfull((fin, emb)),
            wfull((4, tg, emb)),
            wfull((4, tg, emb)),
            wfull((4, tg, emb)),
            wfull((1, emb)),
            wfull((emb, emb)),
            wfull((1, emb)),
        ],
        out_specs=pl.BlockSpec((bn, emb), lambda i: (i, 0)),
        out_shape=jax.ShapeDtypeStruct((n, emb), jnp.float32),
    )(x, s1, s2, mn, mx, deg, wd2, pxf, b1s, b2s, b3s, postb_f, lin_w, lin_b)


def _pool_mlp(h2, batch_b, y_b, fm_W, fm_b, m1_W, m1_b, m2_W, m2_b,
              m3_W, m3_b, m4_Wp, m4_bp, bn):
    n, emb = h2.shape
    d = fm_W.shape[1]
    nsteps = n // bn

    def body(h_ref, b_ref, y_ref, fmw_ref, fmb_ref, w1_ref, b1_ref,
             w2_ref, b2_ref, w3_ref, b3_ref, w4_ref, b4_ref,
             hg_ref, out_ref, loss_ref):
        step = pl.program_id(0)

        @pl.when(step == 0)
        def _():
            hg_ref[...] = jnp.zeros_like(hg_ref)

        gid = lax.broadcasted_iota(jnp.int32, (bn, NUM_GRAPHS), 1)
        oh = jnp.where(b_ref[:, 0:1] == gid, 1.0, 0.0).astype(jnp.float32)
        hg_ref[...] += lax.dot_general(
            oh, h_ref[...], (((0,), (0,)), ((), ())),
            preferred_element_type=jnp.float32)

        @pl.when(step == nsteps - 1)
        def _():
            hg = hg_ref[...]
            embv = jnp.dot(hg, fmw_ref[...],
                           preferred_element_type=jnp.float32) + fmb_ref[...]
            o = jax.nn.relu(jnp.dot(embv, w1_ref[...],
                                    preferred_element_type=jnp.float32)
                            + b1_ref[...])
            o = jax.nn.relu(jnp.dot(o, w2_ref[...],
                                    preferred_element_type=jnp.float32)
                            + b2_ref[...])
            o = jax.nn.relu(jnp.dot(o, w3_ref[...],
                                    preferred_element_type=jnp.float32)
                            + b3_ref[...])
            outp = jnp.dot(o, w4_ref[...],
                           preferred_element_type=jnp.float32) + b4_ref[...]
            out_ref[...] = outp
            diff = outp[:, 0:1] - y_ref[:, 0:1]
            loss_ref[...] = jnp.full((8, 128),
                                     jnp.sum(diff * diff) * (1.0 / NUM_GRAPHS),
                                     jnp.float32)

    wfull = lambda shape: pl.BlockSpec(shape, lambda i: tuple(0 for _ in shape))
    hg_out, out_p, loss_p = pl.pallas_call(
        body,
        grid=(nsteps,),
        in_specs=[
            pl.BlockSpec((bn, emb), lambda i: (i, 0)),
            pl.BlockSpec((bn, 128), lambda i: (i, 0)),
            wfull((NUM_GRAPHS, 128)),
            wfull((emb, d)),
            wfull((1, d)),
            wfull((d, 128)),
            wfull((1, 128)),
            wfull((128, 64)),
            wfull((1, 64)),
            wfull((64, 32)),
            wfull((1, 32)),
            wfull((32, 128)),
            wfull((1, 128)),
        ],
        out_specs=[
            pl.BlockSpec((NUM_GRAPHS, emb), lambda i: (0, 0)),
            pl.BlockSpec((NUM_GRAPHS, 128), lambda i: (0, 0)),
            pl.BlockSpec((8, 128), lambda i: (0, 0)),
        ],
        out_shape=[
            jax.ShapeDtypeStruct((NUM_GRAPHS, emb), jnp.float32),
            jax.ShapeDtypeStruct((NUM_GRAPHS, 128), jnp.float32),
            jax.ShapeDtypeStruct((8, 128), jnp.float32),
        ],
    )(h2, batch_b, y_b, fm_W, fm_b, m1_W, m1_b, m2_W, m2_b, m3_W, m3_b,
      m4_Wp, m4_bp)
    del hg_out
    return out_p, loss_p


def _prep_layer(fin, tgp, eeW, eeb, preW, preb, postW, postb):
    g = preW.shape[2]
    fout = postW.shape[2]
    tg = TOWERS * g
    pad = tgp - tg
    wd = preW[:, :fin, :]
    ws = preW[:, fin:2 * fin, :]
    we = preW[:, 2 * fin:, :]
    wd2 = jnp.pad(wd.transpose(1, 0, 2).reshape(fin, tg), ((0, 0), (0, pad)))
    ws2 = jnp.pad(ws.transpose(1, 0, 2).reshape(fin, tg), ((0, 0), (0, pad)))
    cw = jnp.pad(jnp.einsum('df,tfg->dtg', eeW, we).reshape(4, tg),
                 ((0, 0), (0, pad)))
    cb = jnp.pad((jnp.einsum('f,tfg->tg', eeb, we) + preb).reshape(tg),
                 ((0, pad),))
    px = postW[:, :fin, :]
    pxf = px.transpose(1, 0, 2).reshape(fin, TOWERS * fout)
    p3 = postW[:, fin:, :].reshape(TOWERS, 3, 4, g, fout)
    eye = jnp.eye(TOWERS, dtype=jnp.float32)
    def bmat(s):
        return jnp.pad(jnp.einsum('tkgf,tu->ktguf', p3[:, s], eye)
                       .reshape(4, tg, TOWERS * fout),
                       ((0, 0), (0, pad), (0, 0)))
    b1s, b2s, b3s = bmat(0), bmat(1), bmat(2)
    postb_f = postb.reshape(1, TOWERS * fout)
    return wd2, ws2, cw, cb, pxf, b1s, b2s, b3s, postb_f


def kernel(x, edge_index, edge_attr, batch, y, ee0_W, ee0_b, pre0_W, pre0_b,
           post0_W, post0_b, lin0_W, lin0_b, ee1_W, ee1_b, pre1_W, pre1_b,
           post1_W, post1_b, lin1_W, lin1_b, fm_W, fm_b, m1_W, m1_b,
           m2_W, m2_b, m3_W, m3_b, m4_W, m4_b):
    src = edge_index[0]
    dst = edge_index[1]

    perm = jnp.argsort(dst)
    sdst = dst[perm]
    ssrc = src[perm]
    sattr16 = jnp.pad(edge_attr[perm], ((0, 0), (0, 12)))
    block_starts = jnp.searchsorted(
        sdst, jnp.arange(0, N_NODES + NB, NB, dtype=jnp.int32)
    ).astype(jnp.int32)
    starts_pad = jnp.full((NSP,), N_EDGES, jnp.int32)
    starts_pad = lax.dynamic_update_slice(starts_pad, block_starts, (0,))

    (wd2_0, ws2_0, cw0, cb0, pxf0, b1s0, b2s0, b3s0, pb0) = _prep_layer(
        128, 640, ee0_W, ee0_b, pre0_W, pre0_b, post0_W, post0_b)
    u0 = _mm(x, ws2_0, 1000)
    s1, s2, mn, mx, deg = _sc_edge_kernel(640)(
        u0, ssrc, sdst, sattr16, cw0, cb0, starts_pad)
    s1, s2, mn, mx = (v.reshape(N_NODES, 640) for v in (s1, s2, mn, mx))
    deg = deg.reshape(N_NODES, 16)
    h1 = _node_phase(x, s1, s2, mn, mx, deg, wd2_0, pxf0, b1s0, b2s0, b3s0,
                     pb0, lin0_W, lin0_b.reshape(1, -1), True, 400)

    (wd2_1, ws2_1, cw1, cb1, pxf1, b1s1, b2s1, b3s1, pb1) = _prep_layer(
        160, 896, ee1_W, ee1_b, pre1_W, pre1_b, post1_W, post1_b)
    u1 = _mm(h1, ws2_1, 1000)
    s1, s2, mn, mx, deg = _sc_edge_kernel(896)(
        u1, ssrc, sdst, sattr16, cw1, cb1, starts_pad)
    s1, s2, mn, mx = (v.reshape(N_NODES, 896) for v in (s1, s2, mn, mx))
    deg = deg.reshape(N_NODES, 16)
    h2 = _node_phase(h1, s1, s2, mn, mx, deg, wd2_1, pxf1, b1s1, b2s1, b3s1,
                     pb1, lin1_W, lin1_b.reshape(1, -1), False, 400)

    batch_b = jnp.broadcast_to(batch[:, None], (N_NODES, 128))
    y_b = jnp.broadcast_to(y, (NUM_GRAPHS, 128))
    m4_Wp = jnp.pad(m4_W, ((0, 0), (0, 127)))
    out_p, loss_p = _pool_mlp(
        h2, batch_b, y_b, fm_W, fm_b.reshape(1, -1), m1_W, m1_b.reshape(1, -1),
        m2_W, m2_b.reshape(1, -1), m3_W, m3_b.reshape(1, -1), m4_Wp,
        jnp.broadcast_to(m4_b.reshape(1, 1), (1, 128)), 1000)
    out = out_p[:, 0:1]
    loss = loss_p[0, 0]
    return (out, loss)
